# Initial kernel scaffold; baseline (speedup 1.0000x reference)
import jax
import jax.numpy as jnp
from jax.experimental import pallas as pl


def kernel(x, edge_index, edge_type, W, attn_W, rel_table):
    raise NotImplementedError("write your pallas kernel here")


# speedup vs baseline: 8.9971x; 8.9971x over previous
import functools

import jax
import jax.numpy as jnp
from jax import lax
from jax.experimental import pallas as pl
from jax.experimental.pallas import tpu as pltpu
from jax.experimental.pallas import tpu_sc as plsc

N = 10000
E = 320000
D = 128
NP = 10240
NRELS = 32
NTILES = 32
EPT = E // NTILES
K = 80
NCHUNK = EPT // K
RPS = NP // 16
MRPT = NP // 32

_SC_PARAMS = pltpu.CompilerParams(use_tc_tiling_on_sc=False,
                                  needs_layout_passes=False)


def _tc_front_body(x_ref, wt_ref, a_ref, z_ref, s_ref):
    z = jnp.dot(x_ref[...], wt_ref[...], preferred_element_type=jnp.float32)
    z_ref[...] = z
    s_ref[...] = jnp.dot(z, a_ref[...], preferred_element_type=jnp.float32)


def _tc_front(x, wt, a128):
    blk = 1000
    return pl.pallas_call(
        _tc_front_body,
        grid=(N // blk,),
        in_specs=[
            pl.BlockSpec((blk, D), lambda i: (i, 0)),
            pl.BlockSpec((D, D), lambda i: (0, 0)),
            pl.BlockSpec((D, D), lambda i: (0, 0)),
        ],
        out_specs=[
            pl.BlockSpec((blk, D), lambda i: (i, 0)),
            pl.BlockSpec((blk, D), lambda i: (i, 0)),
        ],
        out_shape=[
            jax.ShapeDtypeStruct((N, D), jnp.float32),
            jax.ShapeDtypeStruct((N, D), jnp.float32),
        ],
    )(x, wt, a128)


def _sc_scatter_body(z_hbm, src_hbm, dst_hbm, et_hbm, s1_hbm, s2_hbm,
                     rel_hbm, hacc_hbm, dacc_hbm,
                     s1_v, s2_v, rel_v, rows_v, srcc_v, dstc_v, etc_v,
                     exrel_v, exc_v, dz_v, h_sh, d_sh, sem):
    cid = lax.axis_index("c")
    sid = lax.axis_index("s")
    wid = cid * 16 + sid
    eb = wid * EPT

    zero16 = jnp.zeros((16,), jnp.float32)
    iota16 = lax.iota(jnp.int32, 16)

    pltpu.sync_copy(s1_hbm, s1_v)
    pltpu.sync_copy(s2_hbm, s2_v)
    pltpu.sync_copy(rel_hbm, rel_v)

    def _zrow(i, _):
        ri = jnp.full((16,), i, jnp.int32)
        for j in range(D // 16):
            plsc.store_scatter(rows_v, [ri, iota16 + 16 * j], zero16)
        return 0
    lax.fori_loop(0, K, _zrow, 0)

    def _zd(i, _):
        dz_v[pl.ds(16 * i, 16)] = zero16
        return 0
    lax.fori_loop(0, RPS // 16, _zd, 0)

    base = sid * RPS
    for k in range(RPS // K):
        pltpu.sync_copy(rows_v, h_sh.at[pl.ds(base + k * K, K)])
    pltpu.sync_copy(dz_v, d_sh.at[pl.ds(base, RPS)])
    plsc.subcore_barrier()

    def _chunk(c, _):
        cb = eb + c * K
        pltpu.sync_copy(src_hbm.at[pl.ds(cb, K)], srcc_v)
        pltpu.sync_copy(dst_hbm.at[pl.ds(cb, K)], dstc_v)
        pltpu.sync_copy(et_hbm.at[pl.ds(cb, K)], etc_v)

        pltpu.async_copy(z_hbm.at[srcc_v], rows_v, sem).wait()

        for g in range(K // 16):
            sl = pl.ds(g * 16, 16)
            sv = srcc_v[sl]
            dv = dstc_v[sl]
            ev = etc_v[sl]
            s1g = plsc.load_gather(s1_v, [sv])
            s2g = plsc.load_gather(s2_v, [dv])
            rg = plsc.load_gather(rel_v, [ev])
            av = s1g + s2g
            lv = jnp.where(av >= 0.0, av, 0.01 * av)
            exv = jnp.exp(lv)
            exrel_v[sl] = exv * rg
            exc_v[sl] = exv

        def _scale(i, _):
            ri = jnp.full((16,), i, jnp.int32)
            cv = plsc.load_gather(exrel_v, [ri])
            for j in range(D // 16):
                cj = iota16 + 16 * j
                v = plsc.load_gather(rows_v, [ri, cj])
                plsc.store_scatter(rows_v, [ri, cj], v * cv)
            return 0
        lax.fori_loop(0, K, _scale, 0)

        pltpu.sync_copy(rows_v, h_sh.at[dstc_v], add=True)
        pltpu.sync_copy(exc_v, d_sh.at[dstc_v], add=True)
        return 0

    lax.fori_loop(0, NCHUNK, _chunk, 0)
    plsc.subcore_barrier()

    ob = cid * NP + base
    pltpu.sync_copy(h_sh.at[pl.ds(base, RPS)], hacc_hbm.at[pl.ds(ob, RPS)])
    pltpu.sync_copy(d_sh.at[pl.ds(base, RPS)], dacc_hbm.at[pl.ds(ob, RPS)])


@functools.partial(
    pl.kernel,
    out_type=(jax.ShapeDtypeStruct((2 * NP, D), jnp.float32),
              jax.ShapeDtypeStruct((2 * NP,), jnp.float32)),
    mesh=plsc.VectorSubcoreMesh(core_axis_name="c", subcore_axis_name="s"),
    compiler_params=_SC_PARAMS,
    scratch_types=[
        pltpu.VMEM((N,), jnp.float32),
        pltpu.VMEM((N,), jnp.float32),
        pltpu.VMEM((NRELS,), jnp.float32),
        pltpu.VMEM((K, D), jnp.float32),
        pltpu.VMEM((K,), jnp.int32),
        pltpu.VMEM((K,), jnp.int32),
        pltpu.VMEM((K,), jnp.int32),
        pltpu.VMEM((K,), jnp.float32),
        pltpu.VMEM((K,), jnp.float32),
        pltpu.VMEM((RPS,), jnp.float32),
        pltpu.VMEM_SHARED((NP, D), jnp.float32),
        pltpu.VMEM_SHARED((NP,), jnp.float32),
        pltpu.SemaphoreType.DMA,
    ],
)
def _sc_scatter(z_hbm, src_hbm, dst_hbm, et_hbm, s1_hbm, s2_hbm, rel_hbm,
                hacc_hbm, dacc_hbm, *scratch):
    _sc_scatter_body(z_hbm, src_hbm, dst_hbm, et_hbm, s1_hbm, s2_hbm,
                     rel_hbm, hacc_hbm, dacc_hbm, *scratch)


def _sc_merge_body(hacc_hbm, dacc_hbm, out_hbm,
                   h0_v, h1_v, d0_v, d1_v):
    cid = lax.axis_index("c")
    sid = lax.axis_index("s")
    wid = cid * 16 + sid
    rb = wid * MRPT

    for k in range(MRPT // K):
        rowb = rb + k * K

        @pl.when(rowb < N)
        def _():
            pltpu.sync_copy(hacc_hbm.at[pl.ds(rowb, K)], h0_v)
            pltpu.sync_copy(hacc_hbm.at[pl.ds(NP + rowb, K)], h1_v)
            pltpu.sync_copy(dacc_hbm.at[pl.ds(rowb, K)], d0_v)
            pltpu.sync_copy(dacc_hbm.at[pl.ds(NP + rowb, K)], d1_v)

            iota16 = lax.iota(jnp.int32, 16)

            def _div(i, _):
                ri = jnp.full((16,), i, jnp.int32)
                d0 = plsc.load_gather(d0_v, [ri])
                d1 = plsc.load_gather(d1_v, [ri])
                dv = d0 + d1
                rv = jnp.where(dv > 0.0, 1.0 / dv, 0.0)
                for j in range(D // 16):
                    cj = iota16 + 16 * j
                    v0 = plsc.load_gather(h0_v, [ri, cj])
                    v1 = plsc.load_gather(h1_v, [ri, cj])
                    plsc.store_scatter(h0_v, [ri, cj], (v0 + v1) * rv)
                return 0
            lax.fori_loop(0, K, _div, 0)

            pltpu.sync_copy(h0_v, out_hbm.at[pl.ds(rowb, K)])


@functools.partial(
    pl.kernel,
    out_type=jax.ShapeDtypeStruct((N, D), jnp.float32),
    mesh=plsc.VectorSubcoreMesh(core_axis_name="c", subcore_axis_name="s"),
    compiler_params=_SC_PARAMS,
    scratch_types=[
        pltpu.VMEM((K, D), jnp.float32),
        pltpu.VMEM((K, D), jnp.float32),
        pltpu.VMEM((K,), jnp.float32),
        pltpu.VMEM((K,), jnp.float32),
    ],
)
def _sc_merge(hacc_hbm, dacc_hbm, out_hbm, *scratch):
    _sc_merge_body(hacc_hbm, dacc_hbm, out_hbm, *scratch)


def kernel(x, edge_index, edge_type, W, attn_W, rel_table):
    wt = W.T
    w1 = attn_W[0, :D]
    w2 = attn_W[0, D:]
    a128 = jnp.zeros((D, D), jnp.float32).at[:, 0].set(w1).at[:, 1].set(w2)

    z, s12 = _tc_front(x, wt, a128)
    s1 = s12[:, 0]
    s2 = s12[:, 1]

    src = edge_index[0]
    dst = edge_index[1]
    rel = rel_table[:, 0]

    hacc, dacc = _sc_scatter(z, src, dst, edge_type, s1, s2, rel)
    return _sc_merge(hacc, dacc)


# speedup vs baseline: 13.0457x; 1.4500x over previous
import functools

import jax
import jax.numpy as jnp
from jax import lax
from jax.experimental import pallas as pl
from jax.experimental.pallas import tpu as pltpu
from jax.experimental.pallas import tpu_sc as plsc

N = 10000
E = 320000
D = 128
NP = 10240
NRELS = 32
NTILES = 32
EPT = E // NTILES
K = 80
NCHUNK = EPT // K
RPS = NP // 16
MRPT = NP // 32

_SC_PARAMS = pltpu.CompilerParams(use_tc_tiling_on_sc=False,
                                  needs_layout_passes=False)


def _tc_front_body(x_ref, wt_ref, a_ref, z_ref, s_ref):
    z = jnp.dot(x_ref[...], wt_ref[...], preferred_element_type=jnp.float32)
    z_ref[...] = z
    s_ref[...] = jnp.dot(z, a_ref[...], preferred_element_type=jnp.float32)


def _tc_front(x, wt, a128):
    blk = 1000
    return pl.pallas_call(
        _tc_front_body,
        grid=(N // blk,),
        in_specs=[
            pl.BlockSpec((blk, D), lambda i: (i, 0)),
            pl.BlockSpec((D, D), lambda i: (0, 0)),
            pl.BlockSpec((D, D), lambda i: (0, 0)),
        ],
        out_specs=[
            pl.BlockSpec((blk, D), lambda i: (i, 0)),
            pl.BlockSpec((blk, D), lambda i: (i, 0)),
        ],
        out_shape=[
            jax.ShapeDtypeStruct((N, D), jnp.float32),
            jax.ShapeDtypeStruct((N, D), jnp.float32),
        ],
    )(x, wt, a128)


BLK = 2000
CPB = BLK // K
NBLK = EPT // BLK


def _sc_scatter_body(z_hbm, src_hbm, dst_hbm, et_hbm, s1_hbm, s2_hbm,
                     rel_hbm, hacc_hbm, dacc_hbm,
                     s1_v, s2_v, rel_v, srcb_v, dstb_v, etb_v,
                     rows0_v, rows1_v, dstc_v, exrel_v, exc_v, dz_v,
                     h_sh, d_sh, sem0, sem1):
    cid = lax.axis_index("c")
    sid = lax.axis_index("s")
    wid = cid * 16 + sid
    eb = wid * EPT

    zero16 = jnp.zeros((16,), jnp.float32)
    iota16 = lax.iota(jnp.int32, 16)

    pltpu.sync_copy(s1_hbm, s1_v)
    pltpu.sync_copy(s2_hbm, s2_v)
    pltpu.sync_copy(rel_hbm, rel_v)

    def _zrow(i, _):
        ri = jnp.full((16,), i, jnp.int32)
        for j in range(D // 16):
            plsc.store_scatter(rows0_v, [ri, iota16 + 16 * j], zero16)
        return 0
    lax.fori_loop(0, K, _zrow, 0)

    def _zd(i, _):
        dz_v[pl.ds(16 * i, 16)] = zero16
        return 0
    lax.fori_loop(0, RPS // 16, _zd, 0)

    base = sid * RPS
    for k in range(RPS // K):
        pltpu.sync_copy(rows0_v, h_sh.at[pl.ds(base + k * K, K)])
    pltpu.sync_copy(dz_v, d_sh.at[pl.ds(base, RPS)])
    plsc.subcore_barrier()

    def _stage_block(b):
        bb = eb + b * BLK
        pltpu.sync_copy(src_hbm.at[pl.ds(bb, BLK)], srcb_v)
        pltpu.sync_copy(dst_hbm.at[pl.ds(bb, BLK)], dstb_v)
        pltpu.sync_copy(et_hbm.at[pl.ds(bb, BLK)], etb_v)

    def _issue_gather(c, rows, sem):
        off = lax.rem(c, CPB) * K
        pltpu.async_copy(z_hbm.at[srcb_v.at[pl.ds(off, K)]], rows, sem)

    def _wait_gather(rows, sem):
        pltpu.make_async_copy(z_hbm.at[pl.ds(0, K)], rows, sem).wait()

    def _scalar(c):
        off = lax.rem(c, CPB) * K
        for g in range(K // 16):
            slb = pl.ds(off + g * 16, 16)
            sl = pl.ds(g * 16, 16)
            sv = srcb_v[slb]
            dv = dstb_v[slb]
            ev = etb_v[slb]
            s1g = plsc.load_gather(s1_v, [sv])
            s2g = plsc.load_gather(s2_v, [dv])
            rg = plsc.load_gather(rel_v, [ev])
            av = s1g + s2g
            lv = jnp.where(av >= 0.0, av, 0.01 * av)
            exv = jnp.exp(lv)
            dstc_v[sl] = dv
            exrel_v[sl] = exv * rg
            exc_v[sl] = exv

    def _scale(rows):
        def body(i, _):
            ri = jnp.full((16,), i, jnp.int32)
            cv = plsc.load_gather(exrel_v, [ri])
            for j in range(D // 16):
                cj = iota16 + 16 * j
                v = plsc.load_gather(rows, [ri, cj])
                plsc.store_scatter(rows, [ri, cj], v * cv)
            return 0
        lax.fori_loop(0, K, body, 0)

    def _process(c, rows_cur, sem_cur, rows_nxt, sem_nxt, issue_next):
        _wait_gather(rows_cur, sem_cur)
        _scalar(c)
        if issue_next:
            @pl.when(lax.rem(c + 1, CPB) == 0)
            def _():
                _stage_block(lax.div(c + 1, CPB))
            _issue_gather(c + 1, rows_nxt, sem_nxt)
        _scale(rows_cur)
        pltpu.sync_copy(rows_cur, h_sh.at[dstc_v], add=True)
        pltpu.sync_copy(exc_v, d_sh.at[dstc_v], add=True)

    _stage_block(0)
    _issue_gather(0, rows0_v, sem0)

    def _pair(c2, _):
        c = 2 * c2
        _process(c, rows0_v, sem0, rows1_v, sem1, True)
        _process(c + 1, rows1_v, sem1, rows0_v, sem0, True)
        return 0
    lax.fori_loop(0, (NCHUNK - 1) // 2, _pair, 0)
    _process(NCHUNK - 1, rows0_v, sem0, None, None, False)

    plsc.subcore_barrier()

    ob = cid * NP + base
    pltpu.sync_copy(h_sh.at[pl.ds(base, RPS)], hacc_hbm.at[pl.ds(ob, RPS)])
    pltpu.sync_copy(d_sh.at[pl.ds(base, RPS)], dacc_hbm.at[pl.ds(ob, RPS)])


@functools.partial(
    pl.kernel,
    out_type=(jax.ShapeDtypeStruct((2 * NP, D), jnp.float32),
              jax.ShapeDtypeStruct((2 * NP,), jnp.float32)),
    mesh=plsc.VectorSubcoreMesh(core_axis_name="c", subcore_axis_name="s"),
    compiler_params=_SC_PARAMS,
    scratch_types=[
        pltpu.VMEM((N,), jnp.float32),
        pltpu.VMEM((N,), jnp.float32),
        pltpu.VMEM((NRELS,), jnp.float32),
        pltpu.VMEM((BLK,), jnp.int32),
        pltpu.VMEM((BLK,), jnp.int32),
        pltpu.VMEM((BLK,), jnp.int32),
        pltpu.VMEM((K, D), jnp.float32),
        pltpu.VMEM((K, D), jnp.float32),
        pltpu.VMEM((K,), jnp.int32),
        pltpu.VMEM((K,), jnp.float32),
        pltpu.VMEM((K,), jnp.float32),
        pltpu.VMEM((RPS,), jnp.float32),
        pltpu.VMEM_SHARED((NP, D), jnp.float32),
        pltpu.VMEM_SHARED((NP,), jnp.float32),
        pltpu.SemaphoreType.DMA,
        pltpu.SemaphoreType.DMA,
    ],
)
def _sc_scatter(z_hbm, src_hbm, dst_hbm, et_hbm, s1_hbm, s2_hbm, rel_hbm,
                hacc_hbm, dacc_hbm, *scratch):
    _sc_scatter_body(z_hbm, src_hbm, dst_hbm, et_hbm, s1_hbm, s2_hbm,
                     rel_hbm, hacc_hbm, dacc_hbm, *scratch)


def _sc_merge_body(hacc_hbm, dacc_hbm, out_hbm,
                   h0_v, h1_v, d0_v, d1_v):
    cid = lax.axis_index("c")
    sid = lax.axis_index("s")
    wid = cid * 16 + sid
    rb = wid * MRPT

    for k in range(MRPT // K):
        rowb = rb + k * K

        @pl.when(rowb < N)
        def _():
            pltpu.sync_copy(hacc_hbm.at[pl.ds(rowb, K)], h0_v)
            pltpu.sync_copy(hacc_hbm.at[pl.ds(NP + rowb, K)], h1_v)
            pltpu.sync_copy(dacc_hbm.at[pl.ds(rowb, K)], d0_v)
            pltpu.sync_copy(dacc_hbm.at[pl.ds(NP + rowb, K)], d1_v)

            iota16 = lax.iota(jnp.int32, 16)

            def _div(i, _):
                ri = jnp.full((16,), i, jnp.int32)
                d0 = plsc.load_gather(d0_v, [ri])
                d1 = plsc.load_gather(d1_v, [ri])
                dv = d0 + d1
                rv = jnp.where(dv > 0.0, 1.0 / dv, 0.0)
                for j in range(D // 16):
                    cj = iota16 + 16 * j
                    v0 = plsc.load_gather(h0_v, [ri, cj])
                    v1 = plsc.load_gather(h1_v, [ri, cj])
                    plsc.store_scatter(h0_v, [ri, cj], (v0 + v1) * rv)
                return 0
            lax.fori_loop(0, K, _div, 0)

            pltpu.sync_copy(h0_v, out_hbm.at[pl.ds(rowb, K)])


@functools.partial(
    pl.kernel,
    out_type=jax.ShapeDtypeStruct((N, D), jnp.float32),
    mesh=plsc.VectorSubcoreMesh(core_axis_name="c", subcore_axis_name="s"),
    compiler_params=_SC_PARAMS,
    scratch_types=[
        pltpu.VMEM((K, D), jnp.float32),
        pltpu.VMEM((K, D), jnp.float32),
        pltpu.VMEM((K,), jnp.float32),
        pltpu.VMEM((K,), jnp.float32),
    ],
)
def _sc_merge(hacc_hbm, dacc_hbm, out_hbm, *scratch):
    _sc_merge_body(hacc_hbm, dacc_hbm, out_hbm, *scratch)


def kernel(x, edge_index, edge_type, W, attn_W, rel_table):
    wt = W.T
    w1 = attn_W[0, :D]
    w2 = attn_W[0, D:]
    a128 = jnp.zeros((D, D), jnp.float32).at[:, 0].set(w1).at[:, 1].set(w2)

    z, s12 = _tc_front(x, wt, a128)
    s1 = s12[:, 0]
    s2 = s12[:, 1]

    src = edge_index[0]
    dst = edge_index[1]
    rel = rel_table[:, 0]

    hacc, dacc = _sc_scatter(z, src, dst, edge_type, s1, s2, rel)
    return _sc_merge(hacc, dacc)


# speedup vs baseline: 13.5564x; 1.0391x over previous
import functools

import jax
import jax.numpy as jnp
from jax import lax
from jax.experimental import pallas as pl
from jax.experimental.pallas import tpu as pltpu
from jax.experimental.pallas import tpu_sc as plsc

N = 10000
E = 320000
D = 128
NP = 10240
NRELS = 32
NTILES = 32
EPT = E // NTILES
K = 80
NCHUNK = EPT // K
RPS = NP // 16
MRPT = NP // 32

_SC_PARAMS = pltpu.CompilerParams(use_tc_tiling_on_sc=False,
                                  needs_layout_passes=False)


def _tc_front_body(x_ref, wt_ref, a_ref, z_ref, s_ref):
    z = jnp.dot(x_ref[...], wt_ref[...], preferred_element_type=jnp.float32)
    z_ref[...] = z
    s_ref[...] = jnp.dot(z, a_ref[...], preferred_element_type=jnp.float32)


def _tc_front(x, wt, a128):
    blk = 1000
    return pl.pallas_call(
        _tc_front_body,
        grid=(N // blk,),
        in_specs=[
            pl.BlockSpec((blk, D), lambda i: (i, 0)),
            pl.BlockSpec((D, D), lambda i: (0, 0)),
            pl.BlockSpec((D, D), lambda i: (0, 0)),
        ],
        out_specs=[
            pl.BlockSpec((blk, D), lambda i: (i, 0)),
            pl.BlockSpec((blk, D), lambda i: (i, 0)),
        ],
        out_shape=[
            jax.ShapeDtypeStruct((N, D), jnp.float32),
            jax.ShapeDtypeStruct((N, D), jnp.float32),
        ],
    )(x, wt, a128)


BLK = 2000
CPB = BLK // K
NBLK = EPT // BLK


def _sc_scatter_body(z_hbm, src_hbm, dst_hbm, et_hbm, s1_hbm, s2_hbm,
                     rel_hbm, hacc_hbm, dacc_hbm,
                     s1_v, s2_v, rel_v, srcb_v, dstb_v, etb_v,
                     rows0_v, rows1_v, dstc0_v, dstc1_v, exrel_v,
                     exc0_v, exc1_v, dz_v, h_sh, d_sh,
                     sem0, sem1, semh0, semh1, semd0, semd1):
    cid = lax.axis_index("c")
    sid = lax.axis_index("s")
    wid = cid * 16 + sid
    eb = wid * EPT

    zero16 = jnp.zeros((16,), jnp.float32)
    iota16 = lax.iota(jnp.int32, 16)

    pltpu.sync_copy(s1_hbm, s1_v)
    pltpu.sync_copy(s2_hbm, s2_v)
    pltpu.sync_copy(rel_hbm, rel_v)

    def _zrow(i, _):
        ri = jnp.full((16,), i, jnp.int32)
        for j in range(D // 16):
            plsc.store_scatter(rows0_v, [ri, iota16 + 16 * j], zero16)
        return 0
    lax.fori_loop(0, K, _zrow, 0)

    def _zd(i, _):
        dz_v[pl.ds(16 * i, 16)] = zero16
        return 0
    lax.fori_loop(0, RPS // 16, _zd, 0)

    base = sid * RPS
    for k in range(RPS // K):
        pltpu.sync_copy(rows0_v, h_sh.at[pl.ds(base + k * K, K)])
    pltpu.sync_copy(dz_v, d_sh.at[pl.ds(base, RPS)])
    plsc.subcore_barrier()

    def _stage_block(b):
        bb = eb + b * BLK
        pltpu.sync_copy(src_hbm.at[pl.ds(bb, BLK)], srcb_v)
        pltpu.sync_copy(dst_hbm.at[pl.ds(bb, BLK)], dstb_v)
        pltpu.sync_copy(et_hbm.at[pl.ds(bb, BLK)], etb_v)

    def _issue_gather(c, rows, sem):
        off = lax.rem(c, CPB) * K
        pltpu.async_copy(z_hbm.at[srcb_v.at[pl.ds(off, K)]], rows, sem)

    def _wait_gather(rows, sem):
        pltpu.make_async_copy(z_hbm.at[pl.ds(0, K)], rows, sem).wait()

    def _scalar(c, dstc, exc):
        off = lax.rem(c, CPB) * K
        for g in range(K // 16):
            slb = pl.ds(off + g * 16, 16)
            sl = pl.ds(g * 16, 16)
            sv = srcb_v[slb]
            dv = dstb_v[slb]
            ev = etb_v[slb]
            s1g = plsc.load_gather(s1_v, [sv])
            s2g = plsc.load_gather(s2_v, [dv])
            rg = plsc.load_gather(rel_v, [ev])
            av = s1g + s2g
            lv = jnp.where(av >= 0.0, av, 0.01 * av)
            exv = jnp.exp(lv)
            dstc[sl] = dv
            exrel_v[sl] = exv * rg
            exc[sl] = exv

    def _scale(rows):
        def body(i, _):
            ri = jnp.full((16,), i, jnp.int32)
            cv = plsc.load_gather(exrel_v, [ri])
            for j in range(D // 16):
                cj = iota16 + 16 * j
                v = plsc.load_gather(rows, [ri, cj])
                plsc.store_scatter(rows, [ri, cj], v * cv)
            return 0
        lax.fori_loop(0, K, body, 0)

    def _wait_scats(rows_cur, dstc, exc, sem_h, sem_d):
        pltpu.make_async_copy(rows_cur, h_sh.at[pl.ds(0, K)], sem_h).wait()
        pltpu.make_async_copy(exc, d_sh.at[pl.ds(0, K)], sem_d).wait()

    def _process(c, rows_cur, sem_cur, dstc, exc, sem_h, sem_d,
                 rows_nxt, sem_nxt, nxt_state, issue_next):
        _wait_gather(rows_cur, sem_cur)
        _scalar(c, dstc, exc)
        if issue_next:
            @pl.when(lax.rem(c + 1, CPB) == 0)
            def _():
                _stage_block(lax.div(c + 1, CPB))

            @pl.when(c >= 1)
            def _():
                _wait_scats(*nxt_state)
            _issue_gather(c + 1, rows_nxt, sem_nxt)
        _scale(rows_cur)

        pltpu.async_copy(rows_cur, h_sh.at[dstc], sem_h, add=True)
        pltpu.async_copy(exc, d_sh.at[dstc], sem_d, add=True)

    _stage_block(0)
    _issue_gather(0, rows0_v, sem0)

    st0 = (rows0_v, dstc0_v, exc0_v, semh0, semd0)
    st1 = (rows1_v, dstc1_v, exc1_v, semh1, semd1)

    def _pair(c2, _):
        c = 2 * c2
        _process(c, rows0_v, sem0, dstc0_v, exc0_v, semh0, semd0,
                 rows1_v, sem1, st1, True)
        _process(c + 1, rows1_v, sem1, dstc1_v, exc1_v, semh1, semd1,
                 rows0_v, sem0, st0, True)
        return 0
    lax.fori_loop(0, (NCHUNK - 1) // 2, _pair, 0)
    _process(NCHUNK - 1, rows0_v, sem0, dstc0_v, exc0_v, semh0, semd0,
             None, None, None, False)
    _wait_scats(*st1)
    _wait_scats(*st0)

    plsc.subcore_barrier()

    ob = cid * NP + base
    pltpu.sync_copy(h_sh.at[pl.ds(base, RPS)], hacc_hbm.at[pl.ds(ob, RPS)])
    pltpu.sync_copy(d_sh.at[pl.ds(base, RPS)], dacc_hbm.at[pl.ds(ob, RPS)])


@functools.partial(
    pl.kernel,
    out_type=(jax.ShapeDtypeStruct((2 * NP, D), jnp.float32),
              jax.ShapeDtypeStruct((2 * NP,), jnp.float32)),
    mesh=plsc.VectorSubcoreMesh(core_axis_name="c", subcore_axis_name="s"),
    compiler_params=_SC_PARAMS,
    scratch_types=[
        pltpu.VMEM((N,), jnp.float32),
        pltpu.VMEM((N,), jnp.float32),
        pltpu.VMEM((NRELS,), jnp.float32),
        pltpu.VMEM((BLK,), jnp.int32),
        pltpu.VMEM((BLK,), jnp.int32),
        pltpu.VMEM((BLK,), jnp.int32),
        pltpu.VMEM((K, D), jnp.float32),
        pltpu.VMEM((K, D), jnp.float32),
        pltpu.VMEM((K,), jnp.int32),
        pltpu.VMEM((K,), jnp.int32),
        pltpu.VMEM((K,), jnp.float32),
        pltpu.VMEM((K,), jnp.float32),
        pltpu.VMEM((K,), jnp.float32),
        pltpu.VMEM((RPS,), jnp.float32),
        pltpu.VMEM_SHARED((NP, D), jnp.float32),
        pltpu.VMEM_SHARED((NP,), jnp.float32),
        pltpu.SemaphoreType.DMA,
        pltpu.SemaphoreType.DMA,
        pltpu.SemaphoreType.DMA,
        pltpu.SemaphoreType.DMA,
        pltpu.SemaphoreType.DMA,
        pltpu.SemaphoreType.DMA,
    ],
)
def _sc_scatter(z_hbm, src_hbm, dst_hbm, et_hbm, s1_hbm, s2_hbm, rel_hbm,
                hacc_hbm, dacc_hbm, *scratch):
    _sc_scatter_body(z_hbm, src_hbm, dst_hbm, et_hbm, s1_hbm, s2_hbm,
                     rel_hbm, hacc_hbm, dacc_hbm, *scratch)


def _sc_merge_body(hacc_hbm, dacc_hbm, out_hbm,
                   h0_v, h1_v, d0_v, d1_v):
    cid = lax.axis_index("c")
    sid = lax.axis_index("s")
    wid = cid * 16 + sid
    rb = wid * MRPT

    for k in range(MRPT // K):
        rowb = rb + k * K

        @pl.when(rowb < N)
        def _():
            pltpu.sync_copy(hacc_hbm.at[pl.ds(rowb, K)], h0_v)
            pltpu.sync_copy(hacc_hbm.at[pl.ds(NP + rowb, K)], h1_v)
            pltpu.sync_copy(dacc_hbm.at[pl.ds(rowb, K)], d0_v)
            pltpu.sync_copy(dacc_hbm.at[pl.ds(NP + rowb, K)], d1_v)

            iota16 = lax.iota(jnp.int32, 16)

            def _div(i, _):
                ri = jnp.full((16,), i, jnp.int32)
                d0 = plsc.load_gather(d0_v, [ri])
                d1 = plsc.load_gather(d1_v, [ri])
                dv = d0 + d1
                rv = jnp.where(dv > 0.0, 1.0 / dv, 0.0)
                for j in range(D // 16):
                    cj = iota16 + 16 * j
                    v0 = plsc.load_gather(h0_v, [ri, cj])
                    v1 = plsc.load_gather(h1_v, [ri, cj])
                    plsc.store_scatter(h0_v, [ri, cj], (v0 + v1) * rv)
                return 0
            lax.fori_loop(0, K, _div, 0)

            pltpu.sync_copy(h0_v, out_hbm.at[pl.ds(rowb, K)])


@functools.partial(
    pl.kernel,
    out_type=jax.ShapeDtypeStruct((N, D), jnp.float32),
    mesh=plsc.VectorSubcoreMesh(core_axis_name="c", subcore_axis_name="s"),
    compiler_params=_SC_PARAMS,
    scratch_types=[
        pltpu.VMEM((K, D), jnp.float32),
        pltpu.VMEM((K, D), jnp.float32),
        pltpu.VMEM((K,), jnp.float32),
        pltpu.VMEM((K,), jnp.float32),
    ],
)
def _sc_merge(hacc_hbm, dacc_hbm, out_hbm, *scratch):
    _sc_merge_body(hacc_hbm, dacc_hbm, out_hbm, *scratch)


def kernel(x, edge_index, edge_type, W, attn_W, rel_table):
    wt = W.T
    w1 = attn_W[0, :D]
    w2 = attn_W[0, D:]
    a128 = jnp.zeros((D, D), jnp.float32).at[:, 0].set(w1).at[:, 1].set(w2)

    z, s12 = _tc_front(x, wt, a128)
    s1 = s12[:, 0]
    s2 = s12[:, 1]

    src = edge_index[0]
    dst = edge_index[1]
    rel = rel_table[:, 0]

    hacc, dacc = _sc_scatter(z, src, dst, edge_type, s1, s2, rel)
    return _sc_merge(hacc, dacc)


# speedup vs baseline: 13.6082x; 1.0038x over previous
import functools

import jax
import jax.numpy as jnp
from jax import lax
from jax.experimental import pallas as pl
from jax.experimental.pallas import tpu as pltpu
from jax.experimental.pallas import tpu_sc as plsc

N = 10000
E = 320000
D = 128
NP = 10240
NRELS = 32
NTILES = 32
EPT = E // NTILES
K = 80
NCHUNK = EPT // K
RPS = NP // 16
MRPT = NP // 32

_SC_PARAMS = pltpu.CompilerParams(use_tc_tiling_on_sc=False,
                                  needs_layout_passes=False)


def _tc_front_body(x_ref, wt_ref, a_ref, z_ref, s_ref):
    z = jnp.dot(x_ref[...], wt_ref[...], preferred_element_type=jnp.float32)
    z_ref[...] = z
    s_ref[...] = jnp.dot(z, a_ref[...], preferred_element_type=jnp.float32)


def _tc_front(x, wt, a128):
    blk = 1000
    return pl.pallas_call(
        _tc_front_body,
        grid=(N // blk,),
        in_specs=[
            pl.BlockSpec((blk, D), lambda i: (i, 0)),
            pl.BlockSpec((D, D), lambda i: (0, 0)),
            pl.BlockSpec((D, D), lambda i: (0, 0)),
        ],
        out_specs=[
            pl.BlockSpec((blk, D), lambda i: (i, 0)),
            pl.BlockSpec((blk, D), lambda i: (i, 0)),
        ],
        out_shape=[
            jax.ShapeDtypeStruct((N, D), jnp.float32),
            jax.ShapeDtypeStruct((N, D), jnp.float32),
        ],
    )(x, wt, a128)


BLK = 2000
CPB = BLK // K
NBLK = EPT // BLK


def _sc_scatter_body(z_hbm, src_hbm, dst_hbm, et_hbm, s1_hbm, s2_hbm,
                     rel_hbm, hacc_hbm, dacc_hbm,
                     s1_v, s2_v, rel_v, srcb_v, dstb_v, etb_v,
                     rows0_v, rows1_v, dstc0_v, dstc1_v, exrel_v,
                     exc0_v, exc1_v, dz_v, h_sh, d_sh,
                     sem0, sem1, semh0, semh1, semd0, semd1):
    cid = lax.axis_index("c")
    sid = lax.axis_index("s")
    wid = cid * 16 + sid
    eb = wid * EPT

    zero16 = jnp.zeros((16,), jnp.float32)
    iota16 = lax.iota(jnp.int32, 16)

    pltpu.sync_copy(s1_hbm, s1_v)
    pltpu.sync_copy(s2_hbm, s2_v)
    pltpu.sync_copy(rel_hbm, rel_v)

    def _zrow(i, _):
        ri = jnp.full((16,), i, jnp.int32)
        for j in range(D // 16):
            plsc.store_scatter(rows0_v, [ri, iota16 + 16 * j], zero16)
        return 0
    lax.fori_loop(0, K, _zrow, 0)

    def _zd(i, _):
        dz_v[pl.ds(16 * i, 16)] = zero16
        return 0
    lax.fori_loop(0, RPS // 16, _zd, 0)

    base = sid * RPS
    for k in range(RPS // K):
        pltpu.sync_copy(rows0_v, h_sh.at[pl.ds(base + k * K, K)])
    pltpu.sync_copy(dz_v, d_sh.at[pl.ds(base, RPS)])
    plsc.subcore_barrier()

    def _stage_block(b):
        bb = eb + b * BLK
        pltpu.sync_copy(src_hbm.at[pl.ds(bb, BLK)], srcb_v)
        pltpu.sync_copy(dst_hbm.at[pl.ds(bb, BLK)], dstb_v)
        pltpu.sync_copy(et_hbm.at[pl.ds(bb, BLK)], etb_v)

    def _issue_gather(c, rows, sem):
        off = lax.rem(c, CPB) * K
        pltpu.async_copy(z_hbm.at[srcb_v.at[pl.ds(off, K)]], rows, sem)

    def _wait_gather(rows, sem):
        pltpu.make_async_copy(z_hbm.at[pl.ds(0, K)], rows, sem).wait()

    def _scalar(c, dstc, exc):
        off = lax.rem(c, CPB) * K
        for g in range(K // 16):
            slb = pl.ds(off + g * 16, 16)
            sl = pl.ds(g * 16, 16)
            sv = srcb_v[slb]
            dv = dstb_v[slb]
            ev = etb_v[slb]
            s1g = plsc.load_gather(s1_v, [sv])
            s2g = plsc.load_gather(s2_v, [dv])
            rg = plsc.load_gather(rel_v, [ev])
            av = s1g + s2g
            lv = jnp.where(av >= 0.0, av, 0.01 * av)
            exv = jnp.exp(lv)
            dstc[sl] = dv
            exrel_v[sl] = exv * rg
            exc[sl] = exv

    def _scale(rows):
        for g in range(K // 16):
            gv = exrel_v[pl.ds(16 * g, 16)]

            def body(l, _):
                ri = jnp.full((16,), 16 * g + l, jnp.int32)
                cv = lax.gather(
                    gv, jnp.full((16, 1), l, jnp.int32),
                    lax.GatherDimensionNumbers(
                        offset_dims=(), collapsed_slice_dims=(0,),
                        start_index_map=(0,)),
                    slice_sizes=(1,),
                    mode=lax.GatherScatterMode.PROMISE_IN_BOUNDS)
                for j in range(D // 16):
                    cj = iota16 + 16 * j
                    v = plsc.load_gather(rows, [ri, cj])
                    plsc.store_scatter(rows, [ri, cj], v * cv)
                return 0
            lax.fori_loop(0, 16, body, 0)

    def _wait_scats(rows_cur, dstc, exc, sem_h, sem_d):
        pltpu.make_async_copy(rows_cur, h_sh.at[pl.ds(0, K)], sem_h).wait()
        pltpu.make_async_copy(exc, d_sh.at[pl.ds(0, K)], sem_d).wait()

    def _process(c, rows_cur, sem_cur, dstc, exc, sem_h, sem_d,
                 rows_nxt, sem_nxt, nxt_state, issue_next):
        _wait_gather(rows_cur, sem_cur)
        _scalar(c, dstc, exc)
        if issue_next:
            @pl.when(lax.rem(c + 1, CPB) == 0)
            def _():
                _stage_block(lax.div(c + 1, CPB))

            @pl.when(c >= 1)
            def _():
                _wait_scats(*nxt_state)
            _issue_gather(c + 1, rows_nxt, sem_nxt)
        _scale(rows_cur)

        pltpu.async_copy(rows_cur, h_sh.at[dstc], sem_h, add=True)
        pltpu.async_copy(exc, d_sh.at[dstc], sem_d, add=True)

    _stage_block(0)
    _issue_gather(0, rows0_v, sem0)

    st0 = (rows0_v, dstc0_v, exc0_v, semh0, semd0)
    st1 = (rows1_v, dstc1_v, exc1_v, semh1, semd1)

    def _pair(c2, _):
        c = 2 * c2
        _process(c, rows0_v, sem0, dstc0_v, exc0_v, semh0, semd0,
                 rows1_v, sem1, st1, True)
        _process(c + 1, rows1_v, sem1, dstc1_v, exc1_v, semh1, semd1,
                 rows0_v, sem0, st0, True)
        return 0
    lax.fori_loop(0, (NCHUNK - 1) // 2, _pair, 0)
    _process(NCHUNK - 1, rows0_v, sem0, dstc0_v, exc0_v, semh0, semd0,
             None, None, None, False)
    _wait_scats(*st1)
    _wait_scats(*st0)

    plsc.subcore_barrier()

    ob = cid * NP + base
    pltpu.sync_copy(h_sh.at[pl.ds(base, RPS)], hacc_hbm.at[pl.ds(ob, RPS)])
    pltpu.sync_copy(d_sh.at[pl.ds(base, RPS)], dacc_hbm.at[pl.ds(ob, RPS)])


@functools.partial(
    pl.kernel,
    out_type=(jax.ShapeDtypeStruct((2 * NP, D), jnp.float32),
              jax.ShapeDtypeStruct((2 * NP,), jnp.float32)),
    mesh=plsc.VectorSubcoreMesh(core_axis_name="c", subcore_axis_name="s"),
    compiler_params=_SC_PARAMS,
    scratch_types=[
        pltpu.VMEM((N,), jnp.float32),
        pltpu.VMEM((N,), jnp.float32),
        pltpu.VMEM((NRELS,), jnp.float32),
        pltpu.VMEM((BLK,), jnp.int32),
        pltpu.VMEM((BLK,), jnp.int32),
        pltpu.VMEM((BLK,), jnp.int32),
        pltpu.VMEM((K, D), jnp.float32),
        pltpu.VMEM((K, D), jnp.float32),
        pltpu.VMEM((K,), jnp.int32),
        pltpu.VMEM((K,), jnp.int32),
        pltpu.VMEM((K,), jnp.float32),
        pltpu.VMEM((K,), jnp.float32),
        pltpu.VMEM((K,), jnp.float32),
        pltpu.VMEM((RPS,), jnp.float32),
        pltpu.VMEM_SHARED((NP, D), jnp.float32),
        pltpu.VMEM_SHARED((NP,), jnp.float32),
        pltpu.SemaphoreType.DMA,
        pltpu.SemaphoreType.DMA,
        pltpu.SemaphoreType.DMA,
        pltpu.SemaphoreType.DMA,
        pltpu.SemaphoreType.DMA,
        pltpu.SemaphoreType.DMA,
    ],
)
def _sc_scatter(z_hbm, src_hbm, dst_hbm, et_hbm, s1_hbm, s2_hbm, rel_hbm,
                hacc_hbm, dacc_hbm, *scratch):
    _sc_scatter_body(z_hbm, src_hbm, dst_hbm, et_hbm, s1_hbm, s2_hbm,
                     rel_hbm, hacc_hbm, dacc_hbm, *scratch)


def _sc_merge_body(hacc_hbm, dacc_hbm, out_hbm,
                   h0_v, h1_v, d0_v, d1_v):
    cid = lax.axis_index("c")
    sid = lax.axis_index("s")
    wid = cid * 16 + sid
    rb = wid * MRPT

    for k in range(MRPT // K):
        rowb = rb + k * K

        @pl.when(rowb < N)
        def _():
            pltpu.sync_copy(hacc_hbm.at[pl.ds(rowb, K)], h0_v)
            pltpu.sync_copy(hacc_hbm.at[pl.ds(NP + rowb, K)], h1_v)
            pltpu.sync_copy(dacc_hbm.at[pl.ds(rowb, K)], d0_v)
            pltpu.sync_copy(dacc_hbm.at[pl.ds(NP + rowb, K)], d1_v)

            iota16 = lax.iota(jnp.int32, 16)

            def _div(i, _):
                ri = jnp.full((16,), i, jnp.int32)
                d0 = plsc.load_gather(d0_v, [ri])
                d1 = plsc.load_gather(d1_v, [ri])
                dv = d0 + d1
                rv = jnp.where(dv > 0.0, 1.0 / dv, 0.0)
                for j in range(D // 16):
                    cj = iota16 + 16 * j
                    v0 = plsc.load_gather(h0_v, [ri, cj])
                    v1 = plsc.load_gather(h1_v, [ri, cj])
                    plsc.store_scatter(h0_v, [ri, cj], (v0 + v1) * rv)
                return 0
            lax.fori_loop(0, K, _div, 0)

            pltpu.sync_copy(h0_v, out_hbm.at[pl.ds(rowb, K)])


@functools.partial(
    pl.kernel,
    out_type=jax.ShapeDtypeStruct((N, D), jnp.float32),
    mesh=plsc.VectorSubcoreMesh(core_axis_name="c", subcore_axis_name="s"),
    compiler_params=_SC_PARAMS,
    scratch_types=[
        pltpu.VMEM((K, D), jnp.float32),
        pltpu.VMEM((K, D), jnp.float32),
        pltpu.VMEM((K,), jnp.float32),
        pltpu.VMEM((K,), jnp.float32),
    ],
)
def _sc_merge(hacc_hbm, dacc_hbm, out_hbm, *scratch):
    _sc_merge_body(hacc_hbm, dacc_hbm, out_hbm, *scratch)


def kernel(x, edge_index, edge_type, W, attn_W, rel_table):
    wt = W.T
    w1 = attn_W[0, :D]
    w2 = attn_W[0, D:]
    a128 = jnp.zeros((D, D), jnp.float32).at[:, 0].set(w1).at[:, 1].set(w2)

    z, s12 = _tc_front(x, wt, a128)
    s1 = s12[:, 0]
    s2 = s12[:, 1]

    src = edge_index[0]
    dst = edge_index[1]
    rel = rel_table[:, 0]

    hacc, dacc = _sc_scatter(z, src, dst, edge_type, s1, s2, rel)
    return _sc_merge(hacc, dacc)


# speedup vs baseline: 26.4391x; 1.9429x over previous
import functools

import jax
import jax.numpy as jnp
from jax import lax
from jax.experimental import pallas as pl
from jax.experimental.pallas import tpu as pltpu
from jax.experimental.pallas import tpu_sc as plsc

N = 10000
E = 320000
D = 128
NP = 10240
NRELS = 32
NTILES = 32
EPT = E // NTILES
K = 80
NCHUNK = EPT // K
RPS = NP // 16
MRPT = NP // 32

_SC_PARAMS = pltpu.CompilerParams(use_tc_tiling_on_sc=False,
                                  needs_layout_passes=False)


def _tc_front_body(x_ref, wt_ref, a_ref, z_ref, s_ref):
    z = jnp.dot(x_ref[...], wt_ref[...], preferred_element_type=jnp.float32)
    z_ref[...] = z
    s_ref[...] = jnp.dot(z, a_ref[...], preferred_element_type=jnp.float32)


def _tc_front(x, wt, a128):
    blk = 1000
    return pl.pallas_call(
        _tc_front_body,
        grid=(N // blk,),
        in_specs=[
            pl.BlockSpec((blk, D), lambda i: (i, 0)),
            pl.BlockSpec((D, D), lambda i: (0, 0)),
            pl.BlockSpec((D, D), lambda i: (0, 0)),
        ],
        out_specs=[
            pl.BlockSpec((blk, D), lambda i: (i, 0)),
            pl.BlockSpec((blk, D), lambda i: (i, 0)),
        ],
        out_shape=[
            jax.ShapeDtypeStruct((N, D), jnp.float32),
            jax.ShapeDtypeStruct((N, D), jnp.float32),
        ],
    )(x, wt, a128)


BLK = 2000
CPB = BLK // K
NBLK = EPT // BLK


def _sc_scatter_body(z_hbm, src_hbm, dst_hbm, et_hbm, s1_hbm, s2_hbm,
                     rel_hbm, hacc_hbm, dacc_hbm,
                     s1_v, s2_v, rel_v, srcb_v, dstb_v, etb_v,
                     rows0_v, rows1_v, dstc0_v, dstc1_v, exrel_v,
                     exc0_v, exc1_v, dz_v, h_sh, d_sh,
                     sem0, sem1, semh0, semh1, semd0, semd1):
    cid = lax.axis_index("c")
    sid = lax.axis_index("s")
    wid = cid * 16 + sid
    eb = wid * EPT

    zero16 = jnp.zeros((16,), jnp.float32)
    iota16 = lax.iota(jnp.int32, 16)

    pltpu.sync_copy(s1_hbm, s1_v)
    pltpu.sync_copy(s2_hbm, s2_v)
    pltpu.sync_copy(rel_hbm, rel_v)

    def _zrow(i, _):
        ri = jnp.full((16,), i, jnp.int32)
        for j in range(D // 16):
            plsc.store_scatter(rows0_v, [ri, iota16 + 16 * j], zero16)
        return 0
    lax.fori_loop(0, K, _zrow, 0)

    def _zd(i, _):
        dz_v[pl.ds(16 * i, 16)] = zero16
        return 0
    lax.fori_loop(0, RPS // 16, _zd, 0)

    base = sid * RPS
    for k in range(RPS // K):
        pltpu.sync_copy(rows0_v, h_sh.at[pl.ds(base + k * K, K)])
    pltpu.sync_copy(dz_v, d_sh.at[pl.ds(base, RPS)])
    plsc.subcore_barrier()

    def _stage_block(b):
        bb = eb + b * BLK
        pltpu.sync_copy(src_hbm.at[pl.ds(bb, BLK)], srcb_v)
        pltpu.sync_copy(dst_hbm.at[pl.ds(bb, BLK)], dstb_v)
        pltpu.sync_copy(et_hbm.at[pl.ds(bb, BLK)], etb_v)

    def _issue_gather(c, rows, sem):
        off = lax.rem(c, CPB) * K
        pltpu.async_copy(z_hbm.at[srcb_v.at[pl.ds(off, K)]], rows, sem)

    def _wait_gather(rows, sem):
        pltpu.make_async_copy(z_hbm.at[pl.ds(0, K)], rows, sem).wait()

    def _scalar(c, dstc, exc):
        off = lax.rem(c, CPB) * K
        for g in range(K // 16):
            slb = pl.ds(off + g * 16, 16)
            sl = pl.ds(g * 16, 16)
            sv = srcb_v[slb]
            dv = dstb_v[slb]
            ev = etb_v[slb]
            s1g = plsc.load_gather(s1_v, [sv])
            s2g = plsc.load_gather(s2_v, [dv])
            rg = plsc.load_gather(rel_v, [ev])
            av = s1g + s2g
            lv = jnp.where(av >= 0.0, av, 0.01 * av)
            exv = jnp.exp(lv)
            dstc[sl] = dv
            exrel_v[sl] = exv * rg
            exc[sl] = exv

    def _scale(rows):
        for g in range(K // 16):
            gv = exrel_v[pl.ds(16 * g, 16)]
            for l in range(16):
                i = 16 * g + l
                cv = lax.gather(
                    gv, jnp.full((16, 1), l, jnp.int32),
                    lax.GatherDimensionNumbers(
                        offset_dims=(), collapsed_slice_dims=(0,),
                        start_index_map=(0,)),
                    slice_sizes=(1,),
                    mode=lax.GatherScatterMode.PROMISE_IN_BOUNDS)
                for j in range(D // 16):
                    sl = pl.ds(16 * j, 16)
                    rows[i, sl] = rows[i, sl] * cv

    def _wait_scats(rows_cur, dstc, exc, sem_h, sem_d):
        pltpu.make_async_copy(rows_cur, h_sh.at[pl.ds(0, K)], sem_h).wait()
        pltpu.make_async_copy(exc, d_sh.at[pl.ds(0, K)], sem_d).wait()

    def _process(c, rows_cur, sem_cur, dstc, exc, sem_h, sem_d,
                 rows_nxt, sem_nxt, nxt_state, issue_next):
        _wait_gather(rows_cur, sem_cur)
        _scalar(c, dstc, exc)
        if issue_next:
            @pl.when(lax.rem(c + 1, CPB) == 0)
            def _():
                _stage_block(lax.div(c + 1, CPB))

            @pl.when(c >= 1)
            def _():
                _wait_scats(*nxt_state)
            _issue_gather(c + 1, rows_nxt, sem_nxt)
        _scale(rows_cur)

        pltpu.async_copy(rows_cur, h_sh.at[dstc], sem_h, add=True)
        pltpu.async_copy(exc, d_sh.at[dstc], sem_d, add=True)

    _stage_block(0)
    _issue_gather(0, rows0_v, sem0)

    st0 = (rows0_v, dstc0_v, exc0_v, semh0, semd0)
    st1 = (rows1_v, dstc1_v, exc1_v, semh1, semd1)

    def _pair(c2, _):
        c = 2 * c2
        _process(c, rows0_v, sem0, dstc0_v, exc0_v, semh0, semd0,
                 rows1_v, sem1, st1, True)
        _process(c + 1, rows1_v, sem1, dstc1_v, exc1_v, semh1, semd1,
                 rows0_v, sem0, st0, True)
        return 0
    lax.fori_loop(0, (NCHUNK - 1) // 2, _pair, 0)
    _process(NCHUNK - 1, rows0_v, sem0, dstc0_v, exc0_v, semh0, semd0,
             None, None, None, False)
    _wait_scats(*st1)
    _wait_scats(*st0)

    plsc.subcore_barrier()

    ob = cid * NP + base
    pltpu.sync_copy(h_sh.at[pl.ds(base, RPS)], hacc_hbm.at[pl.ds(ob, RPS)])
    pltpu.sync_copy(d_sh.at[pl.ds(base, RPS)], dacc_hbm.at[pl.ds(ob, RPS)])


@functools.partial(
    pl.kernel,
    out_type=(jax.ShapeDtypeStruct((2 * NP, D), jnp.float32),
              jax.ShapeDtypeStruct((2 * NP,), jnp.float32)),
    mesh=plsc.VectorSubcoreMesh(core_axis_name="c", subcore_axis_name="s"),
    compiler_params=_SC_PARAMS,
    scratch_types=[
        pltpu.VMEM((N,), jnp.float32),
        pltpu.VMEM((N,), jnp.float32),
        pltpu.VMEM((NRELS,), jnp.float32),
        pltpu.VMEM((BLK,), jnp.int32),
        pltpu.VMEM((BLK,), jnp.int32),
        pltpu.VMEM((BLK,), jnp.int32),
        pltpu.VMEM((K, D), jnp.float32),
        pltpu.VMEM((K, D), jnp.float32),
        pltpu.VMEM((K,), jnp.int32),
        pltpu.VMEM((K,), jnp.int32),
        pltpu.VMEM((K,), jnp.float32),
        pltpu.VMEM((K,), jnp.float32),
        pltpu.VMEM((K,), jnp.float32),
        pltpu.VMEM((RPS,), jnp.float32),
        pltpu.VMEM_SHARED((NP, D), jnp.float32),
        pltpu.VMEM_SHARED((NP,), jnp.float32),
        pltpu.SemaphoreType.DMA,
        pltpu.SemaphoreType.DMA,
        pltpu.SemaphoreType.DMA,
        pltpu.SemaphoreType.DMA,
        pltpu.SemaphoreType.DMA,
        pltpu.SemaphoreType.DMA,
    ],
)
def _sc_scatter(z_hbm, src_hbm, dst_hbm, et_hbm, s1_hbm, s2_hbm, rel_hbm,
                hacc_hbm, dacc_hbm, *scratch):
    _sc_scatter_body(z_hbm, src_hbm, dst_hbm, et_hbm, s1_hbm, s2_hbm,
                     rel_hbm, hacc_hbm, dacc_hbm, *scratch)


def _sc_merge_body(hacc_hbm, dacc_hbm, out_hbm,
                   h0_v, h1_v, d0_v, d1_v):
    cid = lax.axis_index("c")
    sid = lax.axis_index("s")
    wid = cid * 16 + sid
    rb = wid * MRPT

    def _chunk(k, _):
        rowb = rb + k * K

        @pl.when(rowb < N)
        def _():
            pltpu.sync_copy(hacc_hbm.at[pl.ds(rowb, K)], h0_v)
            pltpu.sync_copy(hacc_hbm.at[pl.ds(NP + rowb, K)], h1_v)
            pltpu.sync_copy(dacc_hbm.at[pl.ds(rowb, K)], d0_v)
            pltpu.sync_copy(dacc_hbm.at[pl.ds(NP + rowb, K)], d1_v)

            for g in range(K // 16):
                sl16 = pl.ds(16 * g, 16)
                dv = d0_v[sl16] + d1_v[sl16]
                rg = jnp.where(dv > 0.0, 1.0 / dv, 0.0)
                for l in range(16):
                    i = 16 * g + l
                    rv = lax.gather(
                        rg, jnp.full((16, 1), l, jnp.int32),
                        lax.GatherDimensionNumbers(
                            offset_dims=(), collapsed_slice_dims=(0,),
                            start_index_map=(0,)),
                        slice_sizes=(1,),
                        mode=lax.GatherScatterMode.PROMISE_IN_BOUNDS)
                    for j in range(D // 16):
                        sl = pl.ds(16 * j, 16)
                        h0_v[i, sl] = (h0_v[i, sl] + h1_v[i, sl]) * rv

            pltpu.sync_copy(h0_v, out_hbm.at[pl.ds(rowb, K)])
        return 0

    lax.fori_loop(0, MRPT // K, _chunk, 0)


@functools.partial(
    pl.kernel,
    out_type=jax.ShapeDtypeStruct((N, D), jnp.float32),
    mesh=plsc.VectorSubcoreMesh(core_axis_name="c", subcore_axis_name="s"),
    compiler_params=_SC_PARAMS,
    scratch_types=[
        pltpu.VMEM((K, D), jnp.float32),
        pltpu.VMEM((K, D), jnp.float32),
        pltpu.VMEM((K,), jnp.float32),
        pltpu.VMEM((K,), jnp.float32),
    ],
)
def _sc_merge(hacc_hbm, dacc_hbm, out_hbm, *scratch):
    _sc_merge_body(hacc_hbm, dacc_hbm, out_hbm, *scratch)


def kernel(x, edge_index, edge_type, W, attn_W, rel_table):
    wt = W.T
    w1 = attn_W[0, :D]
    w2 = attn_W[0, D:]
    a128 = jnp.zeros((D, D), jnp.float32).at[:, 0].set(w1).at[:, 1].set(w2)

    z, s12 = _tc_front(x, wt, a128)
    s1 = s12[:, 0]
    s2 = s12[:, 1]

    src = edge_index[0]
    dst = edge_index[1]
    rel = rel_table[:, 0]

    hacc, dacc = _sc_scatter(z, src, dst, edge_type, s1, s2, rel)
    return _sc_merge(hacc, dacc)
